# speedup vs baseline: 1.0057x; 1.0057x over previous
import jax
import jax.numpy as jnp
from jax.experimental import pallas as pl

N = 10000
E = 160000
IN_DIM = 128
EMB = 256
HEADS = 4
LAYERS = 3
EDGE_DIM = 16
B = 64


def _mm_kernel(x_ref, w_ref, b_ref, o_ref):
    o_ref[...] = jax.nn.relu(
        jnp.dot(x_ref[...], w_ref[...], preferred_element_type=jnp.float32)
        + b_ref[...]
    )


def _proj_relu(x, w, b):
    n = x.shape[0]
    blk = 1000
    return pl.pallas_call(
        _mm_kernel,
        grid=(n // blk,),
        in_specs=[
            pl.BlockSpec((blk, x.shape[1]), lambda i: (i, 0)),
            pl.BlockSpec((x.shape[1], w.shape[1]), lambda i: (0, 0)),
            pl.BlockSpec((1, w.shape[1]), lambda i: (0, 0)),
        ],
        out_specs=pl.BlockSpec((blk, w.shape[1]), lambda i: (i, 0)),
        out_shape=jax.ShapeDtypeStruct((n, w.shape[1]), jnp.float32),
    )(x, w, b.reshape(1, -1))


def _gat_conv(x, src, dst, ea, lp):
    n = x.shape[0]
    xh = (x @ lp['W']).reshape(n, HEADS, EMB)
    eh = (ea @ lp['We']).reshape(-1, HEADS, EMB)
    a = (jnp.sum(xh * lp['att_src'], -1)[src]
         + jnp.sum(xh * lp['att_dst'], -1)[dst]
         + jnp.sum(eh * lp['att_e'], -1))
    a = jax.nn.leaky_relu(a, 0.2)
    amax = jax.ops.segment_max(a, dst, num_segments=n)
    amax = jnp.where(jnp.isfinite(amax), amax, 0.0)
    ex = jnp.exp(a - amax[dst])
    den = jax.ops.segment_sum(ex, dst, num_segments=n)
    alpha = ex / (den[dst] + 1e-16)
    out = jax.ops.segment_sum(xh[src] * alpha[..., None], dst, num_segments=n)
    return jnp.mean(out, axis=1) + lp['bias']


def _bn(x, g, b):
    m = jnp.mean(x, 0)
    v = jnp.var(x, 0)
    return (x - m) / jnp.sqrt(v + 1e-5) * g + b


def kernel(x, edge_index, edge_attr, batch, params):
    n = x.shape[0]
    loops = jnp.arange(n, dtype=edge_index.dtype)
    src = jnp.concatenate([edge_index[0], loops])
    dst = jnp.concatenate([edge_index[1], loops])
    ea = jnp.concatenate(
        [edge_attr,
         jnp.broadcast_to(jnp.mean(edge_attr, 0, keepdims=True), (n, EDGE_DIM))], 0)
    h = _proj_relu(x, params['W0'], params['b0'])
    outs = []
    for lp in params['layers']:
        r = h
        h = _gat_conv(h, src, dst, ea, lp)
        h = _bn(h, lp['gamma'], lp['beta'])
        h = jax.nn.relu(h)
        h = r + h
        outs.append(h)
    pooled = [jax.ops.segment_sum(o, batch, num_segments=B) for o in outs]
    zs = jnp.concatenate(pooled, axis=1)
    gates = jax.nn.softmax(zs @ params['Wg'] + params['bg'], axis=1)
    zt = jnp.stack(pooled, axis=1)
    z = jnp.sum(zt * gates[..., None], axis=1)
    return (z, outs[-1])


# speedup vs baseline: 5.8580x; 5.8249x over previous
import jax
import jax.numpy as jnp
from jax.experimental import pallas as pl

N = 10000
NPAD = 10240
E = 160000
IN_DIM = 128
EMB = 256
HEADS = 4
LAYERS = 3
EDGE_DIM = 16
B = 64
ACCW = 1040


def _proj_relu_kernel(x_ref, w_ref, b_ref, o_ref):
    o_ref[...] = jax.nn.relu(
        jnp.dot(x_ref[...], w_ref[...], preferred_element_type=jnp.float32)
        + b_ref[...])


def _proj_relu(x, w, b):
    n = x.shape[0]
    blk = 1000
    return pl.pallas_call(
        _proj_relu_kernel,
        grid=(n // blk,),
        in_specs=[
            pl.BlockSpec((blk, x.shape[1]), lambda i: (i, 0)),
            pl.BlockSpec((x.shape[1], w.shape[1]), lambda i: (0, 0)),
            pl.BlockSpec((1, w.shape[1]), lambda i: (0, 0)),
        ],
        out_specs=pl.BlockSpec((blk, w.shape[1]), lambda i: (i, 0)),
        out_shape=jax.ShapeDtypeStruct((n, w.shape[1]), jnp.float32),
    )(x, w, b.reshape(1, -1))


def _xh_kernel(h_ref, w_ref, xh_ref, s8_ref):
    y = jnp.dot(h_ref[...], w_ref[...], preferred_element_type=jnp.float32)
    xh_ref[...] = y[:, :HEADS * EMB]
    s8_ref[...] = y[:, HEADS * EMB:HEADS * EMB + 8]


def _xh_proj(h, wcat):
    blk = 1000
    return pl.pallas_call(
        _xh_kernel,
        grid=(N // blk,),
        in_specs=[
            pl.BlockSpec((blk, EMB), lambda i: (i, 0)),
            pl.BlockSpec((EMB, HEADS * EMB + 8), lambda i: (0, 0)),
        ],
        out_specs=[
            pl.BlockSpec((blk, HEADS * EMB), lambda i: (i, 0)),
            pl.BlockSpec((blk, 8), lambda i: (i, 0)),
        ],
        out_shape=[
            jax.ShapeDtypeStruct((N, HEADS * EMB), jnp.float32),
            jax.ShapeDtypeStruct((N, 8), jnp.float32),
        ],
    )(h, wcat)


def _ae_kernel(ea_ref, ve_ref, ae_ref, easum_ref):
    i = pl.program_id(0)
    ea = ea_ref[...]
    y = jnp.dot(ea, ve_ref[...], preferred_element_type=jnp.float32)
    for l in range(LAYERS):
        ae_ref[l] = y[:, l * HEADS:(l + 1) * HEADS]
    @pl.when(i == 0)
    def _():
        easum_ref[...] = jnp.zeros_like(easum_ref)
    easum_ref[0:1, :] += jnp.sum(ea, axis=0, keepdims=True)


def _ae_proj(edge_attr, ve_all):
    blk = 4000
    return pl.pallas_call(
        _ae_kernel,
        grid=(E // blk,),
        in_specs=[
            pl.BlockSpec((blk, EDGE_DIM), lambda i: (i, 0)),
            pl.BlockSpec((EDGE_DIM, LAYERS * HEADS), lambda i: (0, 0)),
        ],
        out_specs=[
            pl.BlockSpec((LAYERS, blk, HEADS), lambda i: (0, i, 0)),
            pl.BlockSpec((8, EDGE_DIM), lambda i: (0, 0)),
        ],
        out_shape=[
            jax.ShapeDtypeStruct((LAYERS, E, HEADS), jnp.float32),
            jax.ShapeDtypeStruct((8, EDGE_DIM), jnp.float32),
        ],
    )(edge_attr, ve_all)


def _post_kernel(nd_ref, xh_ref, s8_ref, aux_ref, out_ref, stats_ref):
    i = pl.program_id(0)
    blk = nd_ref.shape[0]
    s_src = s8_ref[:, 0:4]
    s_dst = s8_ref[:, 4:8]
    a = s_src + s_dst + aux_ref[0:1, 0:4]
    a = jnp.maximum(a, 0.2 * a)
    ex_self = jnp.exp(a)
    num = nd_ref[:, :HEADS * EMB].reshape(blk, HEADS, EMB)
    den = nd_ref[:, HEADS * EMB:HEADS * EMB + 4]
    xh = xh_ref[...].reshape(blk, HEADS, EMB)
    den_t = den + ex_self
    msg = (num + ex_self[..., None] * xh) / (den_t[..., None] + 1e-16)
    node_out = jnp.mean(msg, axis=1) + aux_ref[1:2, 0:EMB]
    out_ref[...] = node_out
    @pl.when(i == 0)
    def _():
        stats_ref[...] = jnp.zeros_like(stats_ref)
    stats_ref[0:1, :] += jnp.sum(node_out, axis=0, keepdims=True)
    stats_ref[1:2, :] += jnp.sum(node_out * node_out, axis=0, keepdims=True)


def _post(numden, xh, s8, aux):
    blk = 1000
    return pl.pallas_call(
        _post_kernel,
        grid=(N // blk,),
        in_specs=[
            pl.BlockSpec((blk, ACCW), lambda i: (i, 0)),
            pl.BlockSpec((blk, HEADS * EMB), lambda i: (i, 0)),
            pl.BlockSpec((blk, 8), lambda i: (i, 0)),
            pl.BlockSpec((8, EMB), lambda i: (0, 0)),
        ],
        out_specs=[
            pl.BlockSpec((blk, EMB), lambda i: (i, 0)),
            pl.BlockSpec((8, EMB), lambda i: (0, 0)),
        ],
        out_shape=[
            jax.ShapeDtypeStruct((N, EMB), jnp.float32),
            jax.ShapeDtypeStruct((8, EMB), jnp.float32),
        ],
    )(numden, xh, s8, aux)


def _bnres_kernel(no_ref, stats_ref, r_ref, gb_ref, h_ref):
    m = stats_ref[0:1, :] / N
    v = stats_ref[1:2, :] / N - m * m
    y = (no_ref[...] - m) * jax.lax.rsqrt(v + 1e-5) * gb_ref[0:1, :] + gb_ref[1:2, :]
    h_ref[...] = r_ref[...] + jax.nn.relu(y)


def _bnres(node_out, stats, r, gb):
    blk = 1000
    return pl.pallas_call(
        _bnres_kernel,
        grid=(N // blk,),
        in_specs=[
            pl.BlockSpec((blk, EMB), lambda i: (i, 0)),
            pl.BlockSpec((8, EMB), lambda i: (0, 0)),
            pl.BlockSpec((blk, EMB), lambda i: (i, 0)),
            pl.BlockSpec((8, EMB), lambda i: (0, 0)),
        ],
        out_specs=pl.BlockSpec((blk, EMB), lambda i: (i, 0)),
        out_shape=jax.ShapeDtypeStruct((N, EMB), jnp.float32),
    )(node_out, stats, r, gb)


def _pool_kernel(b_ref, h1_ref, h2_ref, h3_ref, p_ref):
    i = pl.program_id(0)
    mask = (b_ref[...] == jax.lax.broadcasted_iota(jnp.int32, (1, B), 1)
            ).astype(jnp.float32)
    @pl.when(i == 0)
    def _():
        p_ref[...] = jnp.zeros_like(p_ref)
    dn = (((0,), (0,)), ((), ()))
    for l, h_ref in enumerate((h1_ref, h2_ref, h3_ref)):
        p_ref[:, l * EMB:(l + 1) * EMB] += jax.lax.dot_general(
            mask, h_ref[...], dn, preferred_element_type=jnp.float32)


def _pool(batch2d, h1, h2, h3):
    blk = 1000
    return pl.pallas_call(
        _pool_kernel,
        grid=(N // blk,),
        in_specs=[
            pl.BlockSpec((blk, 1), lambda i: (i, 0)),
            pl.BlockSpec((blk, EMB), lambda i: (i, 0)),
            pl.BlockSpec((blk, EMB), lambda i: (i, 0)),
            pl.BlockSpec((blk, EMB), lambda i: (i, 0)),
        ],
        out_specs=pl.BlockSpec((B, LAYERS * EMB), lambda i: (0, 0)),
        out_shape=jax.ShapeDtypeStruct((B, LAYERS * EMB), jnp.float32),
    )(batch2d, h1, h2, h3)


def _gate_kernel(p_ref, wg_ref, bg_ref, z_ref):
    p = p_ref[...]
    logits = jnp.dot(p, wg_ref[...], preferred_element_type=jnp.float32) \
        + bg_ref[0:1, :]
    lane = jax.lax.broadcasted_iota(jnp.int32, (B, 8), 1)
    valid = lane < LAYERS
    neg = jnp.where(valid, logits, -jnp.inf)
    mx = jnp.max(neg, axis=1, keepdims=True)
    e = jnp.where(valid, jnp.exp(logits - mx), 0.0)
    gates = e / jnp.sum(e, axis=1, keepdims=True)
    z = jnp.zeros((B, EMB), jnp.float32)
    for l in range(LAYERS):
        z = z + gates[:, l:l + 1] * p[:, l * EMB:(l + 1) * EMB]
    z_ref[...] = z


def _gate(pooled, wg8, bg8):
    return pl.pallas_call(
        _gate_kernel,
        in_specs=[
            pl.BlockSpec((B, LAYERS * EMB), lambda: (0, 0)),
            pl.BlockSpec((LAYERS * EMB, 8), lambda: (0, 0)),
            pl.BlockSpec((1, 8), lambda: (0, 0)),
        ],
        out_specs=pl.BlockSpec((B, EMB), lambda: (0, 0)),
        out_shape=jax.ShapeDtypeStruct((B, EMB), jnp.float32),
    )(pooled, wg8, bg8)


def _edge_pass(src, dst, ae_l, xh, s8):
    s_src = s8[:, 0:4]
    s_dst = s8[:, 4:8]
    a = s_src[src] + s_dst[dst] + ae_l
    a = jnp.maximum(a, 0.2 * a)
    ex = jnp.exp(a)
    xh4 = xh.reshape(N, HEADS, EMB)
    msg = (xh4[src] * ex[..., None]).reshape(E, HEADS * EMB)
    num = jax.ops.segment_sum(msg, dst, num_segments=N)
    den = jax.ops.segment_sum(ex, dst, num_segments=N)
    numden = jnp.concatenate(
        [num, den, jnp.zeros((N, ACCW - HEADS * EMB - 4), jnp.float32)], axis=1)
    return numden


def kernel(x, edge_index, edge_attr, batch, params):
    src = edge_index[0]
    dst = edge_index[1]

    ve_all = jnp.concatenate(
        [jnp.einsum('dhe,he->dh',
                    lp['We'].reshape(EDGE_DIM, HEADS, EMB), lp['att_e'])
         for lp in params['layers']], axis=1)
    wcats = []
    for lp in params['layers']:
        wr = lp['W'].reshape(EMB, HEADS, EMB)
        wsrc = jnp.einsum('dhe,he->dh', wr, lp['att_src'])
        wdst = jnp.einsum('dhe,he->dh', wr, lp['att_dst'])
        wcats.append(jnp.concatenate([lp['W'], wsrc, wdst], axis=1))

    h = _proj_relu(x, params['W0'], params['b0'])
    ae3, easum = _ae_proj(edge_attr, ve_all)
    ea_mean = easum[0] / E
    ae_loop = (ea_mean @ ve_all.reshape(EDGE_DIM, LAYERS * HEADS)
               ).reshape(LAYERS, HEADS)

    outs = []
    for li, lp in enumerate(params['layers']):
        r = h
        xh, s8 = _xh_proj(h, wcats[li])
        numden = _edge_pass(src, dst, ae3[li], xh, s8)
        aux = jnp.zeros((8, EMB), jnp.float32)
        aux = aux.at[0, 0:4].set(ae_loop[li]).at[1, :].set(lp['bias'])
        node_out, stats = _post(numden, xh, s8, aux)
        gb = jnp.stack([lp['gamma'], lp['beta']], axis=0)
        gb = jnp.concatenate([gb, jnp.zeros((6, EMB), jnp.float32)], axis=0)
        h = _bnres(node_out, stats, r, gb)
        outs.append(h)

    pooled = _pool(batch.reshape(N, 1), outs[0], outs[1], outs[2])
    wg8 = jnp.concatenate(
        [params['Wg'], jnp.zeros((LAYERS * EMB, 8 - LAYERS), jnp.float32)], axis=1)
    bg8 = jnp.concatenate([params['bg'], jnp.zeros((8 - LAYERS,), jnp.float32)]
                          ).reshape(1, 8)
    z = _gate(pooled, wg8, bg8)
    return (z, outs[-1])


# speedup vs baseline: 7.6612x; 1.3078x over previous
import functools

import jax
import jax.numpy as jnp
from jax import lax
from jax.experimental import pallas as pl
from jax.experimental.pallas import tpu as pltpu
from jax.experimental.pallas import tpu_sc as plsc

N = 10000
NPAD = 10240
E = 160000
IN_DIM = 128
EMB = 256
HEADS = 4
LAYERS = 3
EDGE_DIM = 16
B = 64
ACCW = 1152


def _proj_relu_kernel(x_ref, w_ref, b_ref, o_ref):
    o_ref[...] = jax.nn.relu(
        jnp.dot(x_ref[...], w_ref[...], preferred_element_type=jnp.float32)
        + b_ref[...])


def _proj_relu(x, w, b):
    n = x.shape[0]
    blk = 1000
    return pl.pallas_call(
        _proj_relu_kernel,
        grid=(n // blk,),
        in_specs=[
            pl.BlockSpec((blk, x.shape[1]), lambda i: (i, 0)),
            pl.BlockSpec((x.shape[1], w.shape[1]), lambda i: (0, 0)),
            pl.BlockSpec((1, w.shape[1]), lambda i: (0, 0)),
        ],
        out_specs=pl.BlockSpec((blk, w.shape[1]), lambda i: (i, 0)),
        out_shape=jax.ShapeDtypeStruct((n, w.shape[1]), jnp.float32),
    )(x, w, b.reshape(1, -1))


def _xh_kernel(h_ref, w_ref, xh_ref, s8_ref):
    y = jnp.dot(h_ref[...], w_ref[...], preferred_element_type=jnp.float32)
    xh_ref[...] = y[:, :HEADS * EMB]
    s8_ref[...] = y[:, HEADS * EMB:HEADS * EMB + 8]


def _xh_proj(h, wcat):
    blk = 1000
    return pl.pallas_call(
        _xh_kernel,
        grid=(N // blk,),
        in_specs=[
            pl.BlockSpec((blk, EMB), lambda i: (i, 0)),
            pl.BlockSpec((EMB, HEADS * EMB + 8), lambda i: (0, 0)),
        ],
        out_specs=[
            pl.BlockSpec((blk, HEADS * EMB), lambda i: (i, 0)),
            pl.BlockSpec((blk, 8), lambda i: (i, 0)),
        ],
        out_shape=[
            jax.ShapeDtypeStruct((N, HEADS * EMB), jnp.float32),
            jax.ShapeDtypeStruct((N, 8), jnp.float32),
        ],
    )(h, wcat)


def _ae_kernel(ea_ref, ve_ref, ae_ref, easum_ref):
    i = pl.program_id(0)
    ea = ea_ref[...]
    y = jnp.dot(ea, ve_ref[...], preferred_element_type=jnp.float32)
    for l in range(LAYERS):
        ae_ref[l] = y[:, l * HEADS:(l + 1) * HEADS]
    @pl.when(i == 0)
    def _():
        easum_ref[...] = jnp.zeros_like(easum_ref)
    easum_ref[0:1, :] += jnp.sum(ea, axis=0, keepdims=True)


def _ae_proj(edge_attr, ve_all):
    blk = 4000
    return pl.pallas_call(
        _ae_kernel,
        grid=(E // blk,),
        in_specs=[
            pl.BlockSpec((blk, EDGE_DIM), lambda i: (i, 0)),
            pl.BlockSpec((EDGE_DIM, LAYERS * HEADS), lambda i: (0, 0)),
        ],
        out_specs=[
            pl.BlockSpec((LAYERS, blk, HEADS), lambda i: (0, i, 0)),
            pl.BlockSpec((8, EDGE_DIM), lambda i: (0, 0)),
        ],
        out_shape=[
            jax.ShapeDtypeStruct((LAYERS, E, HEADS), jnp.float32),
            jax.ShapeDtypeStruct((8, EDGE_DIM), jnp.float32),
        ],
    )(edge_attr, ve_all)


def _post_kernel(nd_ref, xh_ref, s8_ref, aux_ref, out_ref, stats_ref):
    i = pl.program_id(0)
    blk = nd_ref.shape[0]
    s_src = s8_ref[:, 0:4]
    s_dst = s8_ref[:, 4:8]
    a = s_src + s_dst + aux_ref[0:1, 0:4]
    a = jnp.maximum(a, 0.2 * a)
    ex_self = jnp.exp(a)
    num = nd_ref[:, :HEADS * EMB].reshape(blk, HEADS, EMB)
    den = nd_ref[:, HEADS * EMB:HEADS * EMB + 4]
    xh = xh_ref[...].reshape(blk, HEADS, EMB)
    den_t = den + ex_self
    msg = (num + ex_self[..., None] * xh) / (den_t[..., None] + 1e-16)
    node_out = jnp.mean(msg, axis=1) + aux_ref[1:2, 0:EMB]
    out_ref[...] = node_out
    @pl.when(i == 0)
    def _():
        stats_ref[...] = jnp.zeros_like(stats_ref)
    stats_ref[0:1, :] += jnp.sum(node_out, axis=0, keepdims=True)
    stats_ref[1:2, :] += jnp.sum(node_out * node_out, axis=0, keepdims=True)


def _post(numden, xh, s8, aux):
    blk = 1000
    return pl.pallas_call(
        _post_kernel,
        grid=(N // blk,),
        in_specs=[
            pl.BlockSpec((blk, ACCW), lambda i: (i, 0)),
            pl.BlockSpec((blk, HEADS * EMB), lambda i: (i, 0)),
            pl.BlockSpec((blk, 8), lambda i: (i, 0)),
            pl.BlockSpec((8, EMB), lambda i: (0, 0)),
        ],
        out_specs=[
            pl.BlockSpec((blk, EMB), lambda i: (i, 0)),
            pl.BlockSpec((8, EMB), lambda i: (0, 0)),
        ],
        out_shape=[
            jax.ShapeDtypeStruct((N, EMB), jnp.float32),
            jax.ShapeDtypeStruct((8, EMB), jnp.float32),
        ],
    )(numden, xh, s8, aux)


def _bnres_kernel(no_ref, stats_ref, r_ref, gb_ref, h_ref):
    m = stats_ref[0:1, :] / N
    v = stats_ref[1:2, :] / N - m * m
    y = (no_ref[...] - m) * jax.lax.rsqrt(v + 1e-5) * gb_ref[0:1, :] + gb_ref[1:2, :]
    h_ref[...] = r_ref[...] + jax.nn.relu(y)


def _bnres(node_out, stats, r, gb):
    blk = 1000
    return pl.pallas_call(
        _bnres_kernel,
        grid=(N // blk,),
        in_specs=[
            pl.BlockSpec((blk, EMB), lambda i: (i, 0)),
            pl.BlockSpec((8, EMB), lambda i: (0, 0)),
            pl.BlockSpec((blk, EMB), lambda i: (i, 0)),
            pl.BlockSpec((8, EMB), lambda i: (0, 0)),
        ],
        out_specs=pl.BlockSpec((blk, EMB), lambda i: (i, 0)),
        out_shape=jax.ShapeDtypeStruct((N, EMB), jnp.float32),
    )(node_out, stats, r, gb)


def _pool_kernel(b_ref, h1_ref, h2_ref, h3_ref, p_ref):
    i = pl.program_id(0)
    mask = (b_ref[...] == jax.lax.broadcasted_iota(jnp.int32, (1, B), 1)
            ).astype(jnp.float32)
    @pl.when(i == 0)
    def _():
        p_ref[...] = jnp.zeros_like(p_ref)
    dn = (((0,), (0,)), ((), ()))
    for l, h_ref in enumerate((h1_ref, h2_ref, h3_ref)):
        p_ref[:, l * EMB:(l + 1) * EMB] += jax.lax.dot_general(
            mask, h_ref[...], dn, preferred_element_type=jnp.float32)


def _pool(batch2d, h1, h2, h3):
    blk = 1000
    return pl.pallas_call(
        _pool_kernel,
        grid=(N // blk,),
        in_specs=[
            pl.BlockSpec((blk, 1), lambda i: (i, 0)),
            pl.BlockSpec((blk, EMB), lambda i: (i, 0)),
            pl.BlockSpec((blk, EMB), lambda i: (i, 0)),
            pl.BlockSpec((blk, EMB), lambda i: (i, 0)),
        ],
        out_specs=pl.BlockSpec((B, LAYERS * EMB), lambda i: (0, 0)),
        out_shape=jax.ShapeDtypeStruct((B, LAYERS * EMB), jnp.float32),
    )(batch2d, h1, h2, h3)


def _gate_kernel(p_ref, wg_ref, bg_ref, z_ref):
    p = p_ref[...]
    logits = jnp.dot(p, wg_ref[...], preferred_element_type=jnp.float32) \
        + bg_ref[0:1, :]
    lane = jax.lax.broadcasted_iota(jnp.int32, (B, 8), 1)
    valid = lane < LAYERS
    neg = jnp.where(valid, logits, -jnp.inf)
    mx = jnp.max(neg, axis=1, keepdims=True)
    e = jnp.where(valid, jnp.exp(logits - mx), 0.0)
    gates = e / jnp.sum(e, axis=1, keepdims=True)
    z = jnp.zeros((B, EMB), jnp.float32)
    for l in range(LAYERS):
        z = z + gates[:, l:l + 1] * p[:, l * EMB:(l + 1) * EMB]
    z_ref[...] = z


def _gate(pooled, wg8, bg8):
    return pl.pallas_call(
        _gate_kernel,
        in_specs=[
            pl.BlockSpec((B, LAYERS * EMB), lambda: (0, 0)),
            pl.BlockSpec((LAYERS * EMB, 8), lambda: (0, 0)),
            pl.BlockSpec((1, 8), lambda: (0, 0)),
        ],
        out_specs=pl.BlockSpec((B, EMB), lambda: (0, 0)),
        out_shape=jax.ShapeDtypeStruct((B, EMB), jnp.float32),
    )(pooled, wg8, bg8)


E_PAD = 163840
C_BKT = 1024
NBKT = NPAD // C_BKT
LISTW = 10320

_MESH = plsc.VectorSubcoreMesh(core_axis_name="c", subcore_axis_name="s")


def _sc_ex_records(src_pad, dst_pad, ae_flat, s8_flat):
    ept = E_PAD // 32
    ch = 1024

    @functools.partial(
        pl.kernel,
        out_type=jax.ShapeDtypeStruct((HEADS * E_PAD,), jnp.float32),
        mesh=_MESH,
        compiler_params=pltpu.CompilerParams(needs_layout_passes=False),
        scratch_types=[
            pltpu.VMEM((N * 8,), jnp.float32),
            pltpu.VMEM((ch,), jnp.int32),
            pltpu.VMEM((ch,), jnp.int32),
            pltpu.VMEM((ch * HEADS,), jnp.float32),
            pltpu.VMEM((ch * HEADS,), jnp.float32),
        ],
    )
    def body(src_hbm, dst_hbm, ae_hbm, s8_hbm, ex_hbm, stab, srcb, dstb,
             aeb, exb):
        c = lax.axis_index("c")
        s = lax.axis_index("s")
        base = (s * 2 + c) * ept
        pltpu.sync_copy(s8_hbm, stab)
        lanes = jnp.arange(16, dtype=jnp.int32)

        def chunk(ci, _):
            off = base + ci * ch
            pltpu.sync_copy(src_hbm.at[pl.ds(off, ch)], srcb)
            pltpu.sync_copy(dst_hbm.at[pl.ds(off, ch)], dstb)
            pltpu.sync_copy(ae_hbm.at[pl.ds(off * HEADS, ch * HEADS)], aeb)

            def step(j, _):
                rows = j * 16 + lanes
                sv = srcb[pl.ds(j * 16, 16)]
                dv = dstb[pl.ds(j * 16, 16)]
                valid = (off + rows) < E
                for hd in range(HEADS):
                    colv = jnp.full((16,), hd, jnp.int32)
                    a = (plsc.load_gather(stab, [sv * 8 + colv])
                         + plsc.load_gather(stab, [dv * 8 + colv + 4])
                         + plsc.load_gather(aeb, [rows * HEADS + colv]))
                    a = jnp.maximum(a, 0.2 * a)
                    ex = jnp.where(valid, jnp.exp(a), 0.0)
                    exb[pl.ds(hd * ch + j * 16, 16)] = ex
                return 0

            lax.fori_loop(0, ch // 16, step, 0)
            for hd in range(HEADS):
                pltpu.sync_copy(exb.at[pl.ds(hd * ch, ch)],
                                ex_hbm.at[pl.ds(hd * E_PAD + off, ch)])
            return 0

        lax.fori_loop(0, ept // ch, chunk, 0)

    return body(src_pad, dst_pad, ae_flat, s8_flat)


def _sc_msg_accum(exf, src_pad, dst_pad, xh):
    slice_e = E_PAD // 16
    nbin = 80
    poolw = 11520
    accw_words = 65 * ACCW

    @functools.partial(
        pl.kernel,
        out_type=jax.ShapeDtypeStruct((NPAD * ACCW,), jnp.float32),
        mesh=_MESH,
        compiler_params=pltpu.CompilerParams(needs_layout_passes=False),
        scratch_types=[
            pltpu.VMEM((poolw,), jnp.int32),
            pltpu.VMEM((1024,), jnp.int32),
            pltpu.VMEM((16 * nbin,), jnp.int32),
            pltpu.VMEM((16 * nbin,), jnp.int32),
            pltpu.VMEM((128,), jnp.int32),
            pltpu.VMEM((128,), jnp.int32),
            pltpu.VMEM((4096,), jnp.int32),
            pltpu.VMEM((16,), jnp.int32),
            pltpu.VMEM((16,), jnp.int32),
            pltpu.VMEM((16,), jnp.int32),
            pltpu.VMEM((16,), jnp.int32),
            pltpu.VMEM((64,), jnp.float32),
            pltpu.VMEM((16, HEADS * EMB), jnp.float32),
            pltpu.VMEM((accw_words,), jnp.float32),
            pltpu.VMEM_SHARED((16 * poolw,), jnp.int32),
            pltpu.VMEM_SHARED((4096,), jnp.int32),
            pltpu.SemaphoreType.DMA,
        ],
    )
    def body(ex_hbm, src_hbm, dst_hbm, xh_hbm, nd_hbm, pool, dstb, cnts,
             curs, bstart, bcnt, metav, idb, srcix, dst16, dlb, exb4,
             rows, accf, sbins, smeta, sem):
        c = lax.axis_index("c")
        s = lax.axis_index("s")
        lanes = jnp.arange(16, dtype=jnp.int32)
        sbase = s * slice_e
        zf = jnp.zeros((16,), jnp.float32)
        zi = jnp.zeros((16,), jnp.int32)
        onei = jnp.ones((16,), jnp.int32)

        def zcnt(i, _):
            cnts[pl.ds(i * 16, 16)] = zi
            return 0

        lax.fori_loop(0, nbin, zcnt, 0)

        def binpass(fill):
            def chunk(ci, _):
                off = sbase + ci * 1024
                pltpu.sync_copy(dst_hbm.at[pl.ds(off, 1024)], dstb)

                def step(i, _):
                    dv = plsc.load_gather(dstb, [lanes * 64 + i])
                    gid = off + lanes * 64 + i
                    g = dv >> 6
                    mine = (gid < E) & ((g & 1) == c)
                    b80 = ((g >> 1) & 15) * 5 + (g >> 5)
                    slot = lanes * nbin + b80
                    if fill:
                        cv = plsc.load_gather(curs, [slot])
                        plsc.store_scatter(pool, [cv], gid, mask=mine)
                        plsc.store_scatter(curs, [slot],
                                           cv + mine.astype(jnp.int32))
                    else:
                        plsc.addupdate_scatter(cnts, [slot], onei, mask=mine)
                    return 0

                lax.fori_loop(0, 64, step, 0)
                return 0

            lax.fori_loop(0, slice_e // 1024, chunk, 0)

        binpass(False)

        lane0 = lanes == 0

        def pfx(bb, run):
            cv = plsc.load_gather(cnts, [lanes * nbin + bb])
            pc = jnp.cumsum(cv)
            plsc.store_scatter(curs, [lanes * nbin + bb], run + pc - cv)
            tot = jnp.sum(cv)
            plsc.store_scatter(bstart, [zi + bb], zi + run, mask=lane0)
            plsc.store_scatter(bcnt, [zi + bb], zi + tot, mask=lane0)
            return run + ((tot + 15) & ~15)

        lax.fori_loop(0, nbin, pfx, jnp.int32(0))

        binpass(True)

        pltpu.sync_copy(pool, sbins.at[pl.ds(s * poolw, poolw)])
        pltpu.sync_copy(bstart, smeta.at[pl.ds(s * 256, 128)])
        pltpu.sync_copy(bcnt, smeta.at[pl.ds(s * 256 + 128, 128)])
        plsc.subcore_barrier()
        pltpu.sync_copy(smeta, metav)

        def passbody(p, _):
            mybin = s * 5 + p
            g = (p << 5) | (s << 1) | c

            def zacc(i, _):
                accf[pl.ds(i * 16, 16)] = zf
                return 0

            lax.fori_loop(0, 64 * ACCW // 16, zacc, 0)

            def prodloop(prod, _):
                bs = pl.multiple_of(plsc.load_gather(
                    metav, [zi + (prod * 256 + mybin)])[0], 16)
                ct = plsc.load_gather(
                    metav, [zi + (prod * 256 + 128 + mybin)])[0]

                def batch(i, _):
                    pltpu.sync_copy(
                        sbins.at[pl.ds(pl.multiple_of(
                            prod * poolw + bs + i * 16, 16), 16)], idb)
                    valid = (i * 16 + lanes) < ct
                    idv = jnp.where(valid, idb[...], E)
                    cp1 = pltpu.async_copy(src_hbm.at[idv], srcix, sem)
                    cp2 = pltpu.async_copy(dst_hbm.at[idv], dst16, sem)
                    cps = [pltpu.async_copy(
                        ex_hbm.at[idv + hd * E_PAD],
                        exb4.at[pl.ds(hd * 16, 16)], sem)
                        for hd in range(HEADS)]
                    cp1.wait()
                    cp2.wait()
                    for cp in cps:
                        cp.wait()
                    dlb[...] = jnp.where(valid & (idv < E),
                                         dst16[...] & 63, 64)
                    pltpu.async_copy(xh_hbm.at[srcix], rows, sem).wait()

                    def edge(e, _):
                        dlv = plsc.load_gather(dlb, [zi + e])
                        rowbase = dlv * ACCW
                        for hd in range(HEADS):
                            bc = plsc.load_gather(
                                exb4, [jnp.full((16,), hd * 16, jnp.int32) + e])
                            for j in range(EMB // 16):
                                o = hd * EMB + j * 16
                                plsc.addupdate_scatter(
                                    accf, [rowbase + o + lanes],
                                    rows[e, pl.ds(o, 16)] * bc)
                        denv = plsc.load_gather(
                            exb4, [jnp.minimum(lanes, 3) * 16 + e])
                        denv = jnp.where(lanes < 4, denv, 0.0)
                        plsc.addupdate_scatter(
                            accf, [rowbase + HEADS * EMB + lanes], denv)
                        return 0

                    lax.fori_loop(0, 16, edge, 0)
                    return 0

                lax.fori_loop(0, (ct + 15) >> 4, batch, 0)
                return 0

            lax.fori_loop(0, 16, prodloop, 0)
            pltpu.sync_copy(
                accf.at[pl.ds(0, 64 * ACCW)],
                nd_hbm.at[pl.ds(pl.multiple_of(g * 64 * ACCW, 128),
                                64 * ACCW)])
            return 0

        lax.fori_loop(0, 5, passbody, 0)

    return body(exf, src_pad, dst_pad, xh)


def kernel(x, edge_index, edge_attr, batch, params):
    pad = E_PAD - E
    src_pad = jnp.concatenate([edge_index[0], jnp.zeros((pad,), jnp.int32)])
    dst_pad = jnp.concatenate([edge_index[1], jnp.zeros((pad,), jnp.int32)])

    ve_all = jnp.concatenate(
        [jnp.einsum('dhe,he->dh',
                    lp['We'].reshape(EDGE_DIM, HEADS, EMB), lp['att_e'])
         for lp in params['layers']], axis=1)
    wcats = []
    for lp in params['layers']:
        wr = lp['W'].reshape(EMB, HEADS, EMB)
        wsrc = jnp.einsum('dhe,he->dh', wr, lp['att_src'])
        wdst = jnp.einsum('dhe,he->dh', wr, lp['att_dst'])
        wcats.append(jnp.concatenate([lp['W'], wsrc, wdst], axis=1))

    h = _proj_relu(x, params['W0'], params['b0'])
    ae3, easum = _ae_proj(edge_attr, ve_all)
    ea_mean = easum[0] / E
    ae_loop = (ea_mean @ ve_all.reshape(EDGE_DIM, LAYERS * HEADS)
               ).reshape(LAYERS, HEADS)
    ae3p = jnp.concatenate(
        [ae3, jnp.zeros((LAYERS, pad, HEADS), jnp.float32)], axis=1)

    outs = []
    for li, lp in enumerate(params['layers']):
        r = h
        xh, s8 = _xh_proj(h, wcats[li])
        exf = _sc_ex_records(src_pad, dst_pad, ae3p[li].reshape(-1),
                             s8.reshape(-1))
        numden = _sc_msg_accum(exf, src_pad, dst_pad,
                                xh).reshape(NPAD, ACCW)
        aux = jnp.zeros((8, EMB), jnp.float32)
        aux = aux.at[0, 0:4].set(ae_loop[li]).at[1, :].set(lp['bias'])
        node_out, stats = _post(numden, xh, s8, aux)
        gb = jnp.stack([lp['gamma'], lp['beta']], axis=0)
        gb = jnp.concatenate([gb, jnp.zeros((6, EMB), jnp.float32)], axis=0)
        h = _bnres(node_out, stats, r, gb)
        outs.append(h)

    pooled = _pool(batch.reshape(N, 1), outs[0], outs[1], outs[2])
    wg8 = jnp.concatenate(
        [params['Wg'], jnp.zeros((LAYERS * EMB, 8 - LAYERS), jnp.float32)], axis=1)
    bg8 = jnp.concatenate([params['bg'], jnp.zeros((8 - LAYERS,), jnp.float32)]
                          ).reshape(1, 8)
    z = _gate(pooled, wg8, bg8)
    return (z, outs[-1])
